# baseline (speedup 1.0000x reference)
import functools

import jax
import jax.numpy as jnp
from jax import lax
from jax.experimental import pallas as pl
from jax.experimental.pallas import tpu as pltpu
from jax.experimental.pallas import tpu_sc as plsc

N_NODES = 10000
N_EDGES = 320000
D = 128
DW = 64
L = 16

_info = plsc.get_sparse_core_info()
NC = _info.num_cores
NS = _info.num_subcores
NW = NC * NS
EDGES_PER_WORKER = N_EDGES // NW
CHUNK = 80
NCHUNKS = EDGES_PER_WORKER // CHUNK
NBUF = 3


ROWS_PER_TILE = N_NODES // NS


def _sc_body(ep_hbm, h_hbm, out_hbm,
             pr_s, pr_d, rows_s, rows_d, outb, h_sp, sem_g, sem_o):
    sid = lax.axis_index("s")
    wid = sid * NC + lax.axis_index("c")
    base0 = wid * EDGES_PER_WORKER

    r0 = sid * ROWS_PER_TILE
    for p0 in range(0, ROWS_PER_TILE, CHUNK):
        n = min(CHUNK, ROWS_PER_TILE - p0)
        pltpu.sync_copy(h_hbm.at[pl.ds(r0 + p0, n)],
                        rows_s.at[0, pl.ds(0, n)])
        pltpu.sync_copy(rows_s.at[0, pl.ds(0, n)],
                        h_sp.at[pl.ds(r0 + p0, n)])

    pltpu.sync_copy(ep_hbm.at[0, pl.ds(base0, EDGES_PER_WORKER)], pr_s)
    pltpu.sync_copy(ep_hbm.at[1, pl.ds(base0, EDGES_PER_WORKER)], pr_d)
    plsc.subcore_barrier()

    lane = lax.iota(jnp.int32, L)

    def start_gather(i, b):
        sl = pl.ds(i * CHUNK, CHUNK)
        pltpu.async_copy(h_sp.at[pr_s.at[sl]], rows_s.at[b], sem_g.at[b])
        pltpu.async_copy(h_sp.at[pr_d.at[sl]], rows_d.at[b], sem_g.at[b])

    def wait_gather(i, b):
        sl = pl.ds(i * CHUNK, CHUNK)
        pltpu.make_async_copy(h_sp.at[pr_s.at[sl]], rows_s.at[b],
                              sem_g.at[b]).wait()
        pltpu.make_async_copy(h_sp.at[pr_d.at[sl]], rows_d.at[b],
                              sem_g.at[b]).wait()

    def wait_out(i, b):
        pltpu.make_async_copy(
            outb.at[b], out_hbm.at[pl.ds(base0 + i * CHUNK, CHUNK)],
            sem_o.at[b]).wait()

    for pre in range(NBUF - 1):
        start_gather(pre, pre)

    def chunk_body(i, carry):
        b = i % NBUF

        @pl.when(i + NBUF - 1 < NCHUNKS)
        def _():
            start_gather(i + NBUF - 1, (i + NBUF - 1) % NBUF)

        wait_gather(i, b)

        @pl.when(i >= NBUF)
        def _():
            wait_out(i - NBUF, b)

        def group_body(g, c):
            e0 = g * L

            def edge_body(j, scores):
                e = e0 + j
                acc_h = jnp.zeros((L,), jnp.float32)
                acc_l = jnp.zeros((L,), jnp.float32)
                for k in range(DW // L):
                    x = rows_s[b, e, pl.ds(k * L, L)]
                    y = rows_d[b, e, pl.ds(k * L, L)]
                    p = plsc.bitcast(
                        plsc.bitcast(x, jnp.bfloat16)
                        * plsc.bitcast(y, jnp.bfloat16), jnp.int32)
                    acc_h = acc_h + plsc.bitcast(p, jnp.float32)
                    acc_l = acc_l + plsc.bitcast(p << 16, jnp.float32)
                return jnp.where(lane == j, jnp.sum(acc_h + acc_l), scores)

            scores = lax.fori_loop(0, L, edge_body,
                                   jnp.zeros((L,), jnp.float32), unroll=2)
            outb[b, pl.ds(e0, L)] = scores
            return c

        for g in range(CHUNK // L):
            group_body(g, 0)
        pltpu.async_copy(outb.at[b],
                         out_hbm.at[pl.ds(base0 + i * CHUNK, CHUNK)],
                         sem_o.at[b])
        return carry

    lax.fori_loop(0, NCHUNKS, chunk_body, 0)
    for r in range(NBUF):
        i = NCHUNKS - NBUF + r
        wait_out(i, i % NBUF)


@jax.jit
def _sc_dot(ep, hw):
    mesh = plsc.VectorSubcoreMesh(core_axis_name="c", subcore_axis_name="s")
    f = functools.partial(
        pl.kernel,
        mesh=mesh,
        compiler_params=pltpu.CompilerParams(needs_layout_passes=False,
                                             use_tc_tiling_on_sc=False),
        out_type=jax.ShapeDtypeStruct((N_EDGES,), jnp.float32),
        scratch_types=[
            pltpu.VMEM((EDGES_PER_WORKER,), jnp.int32),
            pltpu.VMEM((EDGES_PER_WORKER,), jnp.int32),
            pltpu.VMEM((NBUF, CHUNK, DW), jnp.int32),
            pltpu.VMEM((NBUF, CHUNK, DW), jnp.int32),
            pltpu.VMEM((NBUF, CHUNK), jnp.float32),
            pltpu.VMEM_SHARED((N_NODES, DW), jnp.int32),
            pltpu.SemaphoreType.DMA((NBUF,)),
            pltpu.SemaphoreType.DMA((NBUF,)),
        ],
    )(_sc_body)
    return f(ep, hw)


def kernel(h, edge_index):
    ep = edge_index.astype(jnp.int32)
    u = lax.bitcast_convert_type(h, jnp.uint32)
    r = (u + jnp.uint32(0x7FFF) + ((u >> 16) & jnp.uint32(1))) >> 16
    hw = lax.bitcast_convert_type(r[:, :DW] | (r[:, DW:] << 16), jnp.int32)
    return _sc_dot(ep, hw).reshape(N_EDGES, 1)


# speedup vs baseline: 1.0714x; 1.0714x over previous
import functools

import jax
import jax.numpy as jnp
from jax import lax
from jax.experimental import pallas as pl
from jax.experimental.pallas import tpu as pltpu
from jax.experimental.pallas import tpu_sc as plsc

N_NODES = 10000
N_EDGES = 320000
D = 128
DW = 64
L = 16

_info = plsc.get_sparse_core_info()
NC = _info.num_cores
NS = _info.num_subcores
NW = NC * NS
EDGES_PER_WORKER = N_EDGES // NW
CHUNK = 80
NCHUNKS = EDGES_PER_WORKER // CHUNK
NBUF = 4


ROWS_PER_TILE = N_NODES // NS


def _sc_body(ep_hbm, h_hbm, out_hbm,
             pr_s, pr_d, rows_s, rows_d, outb, h_sp, sem_g, sem_o):
    sid = lax.axis_index("s")
    wid = sid * NC + lax.axis_index("c")
    base0 = wid * EDGES_PER_WORKER

    r0 = sid * ROWS_PER_TILE
    pltpu.sync_copy(h_hbm.at[pl.ds(r0, ROWS_PER_TILE)],
                    h_sp.at[pl.ds(r0, ROWS_PER_TILE)])

    pltpu.sync_copy(ep_hbm.at[0, pl.ds(base0, EDGES_PER_WORKER)], pr_s)
    pltpu.sync_copy(ep_hbm.at[1, pl.ds(base0, EDGES_PER_WORKER)], pr_d)
    plsc.subcore_barrier()

    lane = lax.iota(jnp.int32, L)

    def start_gather(i, b):
        sl = pl.ds(i * CHUNK, CHUNK)
        pltpu.async_copy(h_sp.at[pr_s.at[sl]], rows_s.at[b], sem_g.at[b])
        pltpu.async_copy(h_sp.at[pr_d.at[sl]], rows_d.at[b], sem_g.at[b])

    def wait_gather(i, b):
        sl = pl.ds(i * CHUNK, CHUNK)
        pltpu.make_async_copy(h_sp.at[pr_s.at[sl]], rows_s.at[b],
                              sem_g.at[b]).wait()
        pltpu.make_async_copy(h_sp.at[pr_d.at[sl]], rows_d.at[b],
                              sem_g.at[b]).wait()

    def wait_out(i, b):
        pltpu.make_async_copy(
            outb.at[b], out_hbm.at[pl.ds(base0 + i * CHUNK, CHUNK)],
            sem_o.at[b]).wait()

    for pre in range(NBUF - 1):
        start_gather(pre, pre)

    def chunk_body(i, carry):
        b = i % NBUF

        @pl.when(i + NBUF - 1 < NCHUNKS)
        def _():
            start_gather(i + NBUF - 1, (i + NBUF - 1) % NBUF)

        wait_gather(i, b)

        @pl.when(i >= NBUF)
        def _():
            wait_out(i - NBUF, b)

        def group_body(g, c):
            e0 = g * L

            def edge_body(j, scores):
                e = e0 + j
                acc_h = jnp.zeros((L,), jnp.float32)
                acc_l = jnp.zeros((L,), jnp.float32)
                for k in range(DW // L):
                    x = rows_s[b, e, pl.ds(k * L, L)]
                    y = rows_d[b, e, pl.ds(k * L, L)]
                    p = plsc.bitcast(
                        plsc.bitcast(x, jnp.bfloat16)
                        * plsc.bitcast(y, jnp.bfloat16), jnp.int32)
                    acc_h = acc_h + plsc.bitcast(p, jnp.float32)
                    acc_l = acc_l + plsc.bitcast(p << 16, jnp.float32)
                return jnp.where(lane == j, jnp.sum(acc_h + acc_l), scores)

            scores = lax.fori_loop(0, L, edge_body,
                                   jnp.zeros((L,), jnp.float32), unroll=2)
            outb[b, pl.ds(e0, L)] = scores
            return c

        for g in range(CHUNK // L):
            group_body(g, 0)
        pltpu.async_copy(outb.at[b],
                         out_hbm.at[pl.ds(base0 + i * CHUNK, CHUNK)],
                         sem_o.at[b])
        return carry

    lax.fori_loop(0, NCHUNKS, chunk_body, 0)
    for r in range(NBUF):
        i = NCHUNKS - NBUF + r
        wait_out(i, i % NBUF)


@jax.jit
def _sc_dot(ep, hw):
    mesh = plsc.VectorSubcoreMesh(core_axis_name="c", subcore_axis_name="s")
    f = functools.partial(
        pl.kernel,
        mesh=mesh,
        compiler_params=pltpu.CompilerParams(needs_layout_passes=False,
                                             use_tc_tiling_on_sc=False),
        out_type=jax.ShapeDtypeStruct((N_EDGES,), jnp.float32),
        scratch_types=[
            pltpu.VMEM((EDGES_PER_WORKER,), jnp.int32),
            pltpu.VMEM((EDGES_PER_WORKER,), jnp.int32),
            pltpu.VMEM((NBUF, CHUNK, DW), jnp.int32),
            pltpu.VMEM((NBUF, CHUNK, DW), jnp.int32),
            pltpu.VMEM((NBUF, CHUNK), jnp.float32),
            pltpu.VMEM_SHARED((N_NODES, DW), jnp.int32),
            pltpu.SemaphoreType.DMA((NBUF,)),
            pltpu.SemaphoreType.DMA((NBUF,)),
        ],
    )(_sc_body)
    return f(ep, hw)


def kernel(h, edge_index):
    ep = edge_index.astype(jnp.int32)
    u = lax.bitcast_convert_type(h, jnp.uint32)
    r = (u + jnp.uint32(0x7FFF) + ((u >> 16) & jnp.uint32(1))) >> 16
    hw = lax.bitcast_convert_type(r[:, :DW] | (r[:, DW:] << 16), jnp.int32)
    return _sc_dot(ep, hw).reshape(N_EDGES, 1)
